# Initial kernel scaffold; baseline (speedup 1.0000x reference)
import jax
import jax.numpy as jnp
from jax.experimental import pallas as pl


def kernel(points, random_key, W1, b1, scale1, bias1, mean1, var1, W2, b2, scale2, bias2, mean2, var2):
    raise NotImplementedError("write your pallas kernel here")


# speedup vs baseline: 48.9930x; 48.9930x over previous
import jax
import jax.numpy as jnp
from jax.experimental import pallas as pl
from jax.experimental.pallas import tpu as pltpu

_N = 16384
_M = 256
_K = 32
_EPS = 1e-5
_ROWS = 8
_CHUNKS = _M // _ROWS

_NEG_INF = float('-inf')


def _fps_kernel(x_ref, y_ref, z_ref, id0_ref, ids_ref):
    x = x_ref[...]
    y = y_ref[...]
    z = z_ref[...]
    rows = jax.lax.broadcasted_iota(jnp.int32, (128, 128), 0)
    cols = jax.lax.broadcasted_iota(jnp.int32, (128, 128), 1)
    lin = rows * 128 + cols

    def coords_of(idx):
        sel = lin == idx
        cx = jnp.sum(jnp.where(sel, x, 0.0))
        cy = jnp.sum(jnp.where(sel, y, 0.0))
        cz = jnp.sum(jnp.where(sel, z, 0.0))
        return cx, cy, cz

    id0 = id0_ref[0]
    ids_ref[0] = id0
    qx, qy, qz = coords_of(id0)
    dist0 = jnp.full((128, 128), jnp.inf, jnp.float32)

    def body(k, carry):
        dist, qx, qy, qz, cur = carry
        dx = x - qx
        dy = y - qy
        dz = z - qz
        d = (dx * dx + dz * dz) + dy * dy
        dist = jnp.where(d < dist, d, dist)
        dist = jnp.where(lin == cur, _NEG_INF, dist)
        m = jnp.max(dist)
        idx = jnp.min(jnp.where(dist == m, lin, _N))
        nx, ny, nz = coords_of(idx)
        ids_ref[k + 1] = idx
        return dist, nx, ny, nz, idx

    jax.lax.fori_loop(0, _M - 1, body, (dist0, qx, qy, qz, id0))


def _knn_kernel(ids_ref, x_ref, y_ref, z_ref, ptb_ref, out_ref):
    i = pl.program_id(0)
    x = x_ref[...]
    y = y_ref[...]
    z = z_ref[...]
    pp = (x * x + z * z) + y * y

    riota = jax.lax.broadcasted_iota(jnp.int32, (_ROWS, 1), 0)
    liota = jax.lax.broadcasted_iota(jnp.int32, (1, 128), 1)

    cx = jnp.zeros((_ROWS, 1), jnp.float32)
    cy = jnp.zeros((_ROWS, 1), jnp.float32)
    cz = jnp.zeros((_ROWS, 1), jnp.float32)
    cid = jnp.zeros((_ROWS, 1), jnp.float32)
    for r in range(_ROWS):
        idr = ids_ref[i * _ROWS + r]
        row = idr // 128
        lane = idr % 128
        sel = liota == lane
        gx = jnp.sum(jnp.where(sel, x_ref[pl.ds(row, 1), :], 0.0))
        gy = jnp.sum(jnp.where(sel, y_ref[pl.ds(row, 1), :], 0.0))
        gz = jnp.sum(jnp.where(sel, z_ref[pl.ds(row, 1), :], 0.0))
        cx = jnp.where(riota == r, gx, cx)
        cy = jnp.where(riota == r, gy, cy)
        cz = jnp.where(riota == r, gz, cz)
        cid = jnp.where(riota == r, idr.astype(jnp.float32), cid)

    cc = (cx * cx + cz * cz) + cy * cy

    cmat = jnp.where(liota == 0, cx, jnp.where(liota == 1, cy,
                     jnp.where(liota == 2, cz, 0.0)))
    mm = jnp.dot(cmat.astype(jnp.bfloat16), ptb_ref[...],
                 preferred_element_type=jnp.float32)

    wv_parts = []
    wx_parts = []
    wy_parts = []
    wz_parts = []
    for u in range(8):
        m4 = u % 4
        hi = u // 4
        wv = None
        for t in range(16):
            c = m4 + 4 * t + 64 * hi
            s = -((cc + pp[c:c + 1, :]) - 2.0 * mm[:, c * 128:(c + 1) * 128])
            sx = x[c:c + 1, :]
            sy = y[c:c + 1, :]
            sz = z[c:c + 1, :]
            if wv is None:
                wv = s
                wx = jnp.broadcast_to(sx, s.shape)
                wy = jnp.broadcast_to(sy, s.shape)
                wz = jnp.broadcast_to(sz, s.shape)
            else:
                cond = s > wv
                wv = jnp.where(cond, s, wv)
                wx = jnp.where(cond, sx, wx)
                wy = jnp.where(cond, sy, wy)
                wz = jnp.where(cond, sz, wz)
        wv_parts.append(wv)
        wx_parts.append(wx)
        wy_parts.append(wy)
        wz_parts.append(wz)
    wv = jnp.concatenate(wv_parts, axis=1)
    wx = jnp.concatenate(wx_parts, axis=1)
    wy = jnp.concatenate(wy_parts, axis=1)
    wz = jnp.concatenate(wz_parts, axis=1)

    biota = jax.lax.broadcasted_iota(jnp.int32, (_ROWS, 1024), 1)
    kiota = jax.lax.broadcasted_iota(jnp.int32, (_ROWS, _K), 1)
    fx = jnp.zeros((_ROWS, _K), jnp.float32)
    fy = jnp.zeros((_ROWS, _K), jnp.float32)
    fz = jnp.zeros((_ROWS, _K), jnp.float32)
    for k in range(_K):
        m = jnp.max(wv, axis=1, keepdims=True)
        amin = jnp.min(jnp.where(wv == m, biota, 4096), axis=1, keepdims=True)
        sel = biota == amin
        gx = jnp.sum(jnp.where(sel, wx, 0.0), axis=1, keepdims=True)
        gy = jnp.sum(jnp.where(sel, wy, 0.0), axis=1, keepdims=True)
        gz = jnp.sum(jnp.where(sel, wz, 0.0), axis=1, keepdims=True)
        fx = jnp.where(kiota == k, gx, fx)
        fy = jnp.where(kiota == k, gy, fy)
        fz = jnp.where(kiota == k, gz, fz)
        wv = jnp.where(sel, _NEG_INF, wv)

    out_ref[0, :, :] = fx - cid
    out_ref[1, :, :] = fy - cid
    out_ref[2, :, :] = fz - cid
    out_ref[3, :, :] = fx
    out_ref[4, :, :] = fy
    out_ref[5, :, :] = fz


def _mlp_kernel(f_ref, w1_ref, b1_ref, s1_ref, t1_ref, m1_ref, v1_ref,
                w2_ref, b2_ref, s2_ref, t2_ref, m2_ref, v2_ref, out_ref):
    yy = jnp.dot(f_ref[...], w1_ref[...],
                 preferred_element_type=jnp.float32) + b1_ref[...]
    yy = (yy - m1_ref[...]) / jnp.sqrt(v1_ref[...] + _EPS) * s1_ref[...] + t1_ref[...]
    yy = jnp.maximum(yy, 0.0)
    yy = jnp.dot(yy, w2_ref[...],
                 preferred_element_type=jnp.float32) + b2_ref[...]
    yy = (yy - m2_ref[...]) / jnp.sqrt(v2_ref[...] + _EPS) * s2_ref[...] + t2_ref[...]
    out_ref[...] = jnp.maximum(yy, 0.0)


def kernel(points, random_key, W1, b1, scale1, bias1, mean1, var1,
           W2, b2, scale2, bias2, mean2, var2):
    pts = points[:, :3]
    px = pts[:, 0].reshape(128, 128)
    py = pts[:, 1].reshape(128, 128)
    pz = pts[:, 2].reshape(128, 128)

    id0 = jax.random.choice(random_key, _N, replace=False)
    id0 = jnp.asarray(id0, jnp.int32).reshape(1)

    ids = pl.pallas_call(
        _fps_kernel,
        in_specs=[pl.BlockSpec((128, 128), lambda: (0, 0))] * 3 +
                 [pl.BlockSpec(memory_space=pltpu.SMEM)],
        out_specs=pl.BlockSpec(memory_space=pltpu.SMEM),
        out_shape=jax.ShapeDtypeStruct((_M,), jnp.int32),
    )(px, py, pz, id0)

    ptb = jnp.zeros((128, _N), jnp.float32).at[:3, :].set(pts.T)
    ptb = ptb.astype(jnp.bfloat16)

    feats6 = pl.pallas_call(
        _knn_kernel,
        grid=(_CHUNKS,),
        in_specs=[pl.BlockSpec(memory_space=pltpu.SMEM),
                  pl.BlockSpec((128, 128), lambda i: (0, 0)),
                  pl.BlockSpec((128, 128), lambda i: (0, 0)),
                  pl.BlockSpec((128, 128), lambda i: (0, 0)),
                  pl.BlockSpec((128, _N), lambda i: (0, 0))],
        out_specs=pl.BlockSpec((6, _ROWS, _K), lambda i: (0, i, 0)),
        out_shape=jax.ShapeDtypeStruct((6, _M, _K), jnp.float32),
    )(ids, px, py, pz, ptb)

    feats = jnp.transpose(feats6, (1, 2, 0)).reshape(_M * _K, 6)
    feats = jnp.concatenate(
        [feats, jnp.zeros((_M * _K, 2), jnp.float32)], axis=1)

    w1p = jnp.zeros((8, 128), jnp.float32).at[:6, :].set(W1)
    row = lambda v: v.reshape(1, 128)

    BR = 1024
    out = pl.pallas_call(
        _mlp_kernel,
        grid=(_M * _K // BR,),
        in_specs=[pl.BlockSpec((BR, 8), lambda i: (i, 0))] +
                 [pl.BlockSpec((8, 128), lambda i: (0, 0))] +
                 [pl.BlockSpec((1, 128), lambda i: (0, 0))] * 5 +
                 [pl.BlockSpec((128, 128), lambda i: (0, 0))] +
                 [pl.BlockSpec((1, 128), lambda i: (0, 0))] * 5,
        out_specs=pl.BlockSpec((BR, 128), lambda i: (i, 0)),
        out_shape=jax.ShapeDtypeStruct((_M * _K, 128), jnp.float32),
    )(feats, w1p, row(b1), row(scale1), row(bias1), row(mean1), row(var1),
      W2, row(b2), row(scale2), row(bias2), row(mean2), row(var2))

    return out.reshape(_M, _K, 128)


# speedup vs baseline: 85.8642x; 1.7526x over previous
import jax
import jax.numpy as jnp
from jax.experimental import pallas as pl
from jax.experimental.pallas import tpu as pltpu

_N = 16384
_M = 256
_K = 32
_EPS = 1e-5
_ROWS = 32
_CHUNKS = _M // _ROWS

_NEG_INF = float('-inf')


def _fps_kernel(x_ref, y_ref, z_ref, id0_ref, ids_ref):
    x = x_ref[...]
    y = y_ref[...]
    z = z_ref[...]
    rows = jax.lax.broadcasted_iota(jnp.int32, (128, 128), 0)
    cols = jax.lax.broadcasted_iota(jnp.int32, (128, 128), 1)
    lin = rows * 128 + cols
    liota = jax.lax.broadcasted_iota(jnp.int32, (1, 128), 1)

    def coords_of(idx):
        row = idx // 128
        sel = liota == (idx % 128)
        cx = jnp.sum(jnp.where(sel, x_ref[pl.ds(row, 1), :], 0.0))
        cy = jnp.sum(jnp.where(sel, y_ref[pl.ds(row, 1), :], 0.0))
        cz = jnp.sum(jnp.where(sel, z_ref[pl.ds(row, 1), :], 0.0))
        return cx, cy, cz

    id0 = id0_ref[0]
    ids_ref[0] = id0
    qx, qy, qz = coords_of(id0)
    dist0 = jnp.full((128, 128), jnp.inf, jnp.float32)

    def body(k, carry):
        dist, qx, qy, qz, cur = carry
        dx = x - qx
        dy = y - qy
        dz = z - qz
        d = (dx * dx + dz * dz) + dy * dy
        dist = jnp.where(d < dist, d, dist)
        dist = jnp.where(lin == cur, _NEG_INF, dist)
        m = jnp.max(dist)
        idx = jnp.min(jnp.where(dist == m, lin, _N))
        nx, ny, nz = coords_of(idx)
        ids_ref[k + 1] = idx
        return dist, nx, ny, nz, idx

    jax.lax.fori_loop(0, _M - 1, body, (dist0, qx, qy, qz, id0))


def _knn_kernel(ids_ref, x_ref, y_ref, z_ref, ptb_ref, out_ref):
    i = pl.program_id(0)
    x = x_ref[...]
    y = y_ref[...]
    z = z_ref[...]
    pp = (x * x + z * z) + y * y

    riota = jax.lax.broadcasted_iota(jnp.int32, (_ROWS, 1), 0)
    liota = jax.lax.broadcasted_iota(jnp.int32, (1, 128), 1)

    cx = jnp.zeros((_ROWS, 1), jnp.float32)
    cy = jnp.zeros((_ROWS, 1), jnp.float32)
    cz = jnp.zeros((_ROWS, 1), jnp.float32)
    cid = jnp.zeros((_ROWS, 1), jnp.float32)
    for r in range(_ROWS):
        idr = ids_ref[i * _ROWS + r]
        row = idr // 128
        lane = idr % 128
        sel = liota == lane
        gx = jnp.sum(jnp.where(sel, x_ref[pl.ds(row, 1), :], 0.0))
        gy = jnp.sum(jnp.where(sel, y_ref[pl.ds(row, 1), :], 0.0))
        gz = jnp.sum(jnp.where(sel, z_ref[pl.ds(row, 1), :], 0.0))
        cx = jnp.where(riota == r, gx, cx)
        cy = jnp.where(riota == r, gy, cy)
        cz = jnp.where(riota == r, gz, cz)
        cid = jnp.where(riota == r, idr.astype(jnp.float32), cid)

    cc = (cx * cx + cz * cz) + cy * cy

    cmat = jnp.where(liota == 0, cx, jnp.where(liota == 1, cy,
                     jnp.where(liota == 2, cz, 0.0)))
    mm = jnp.dot(cmat.astype(jnp.bfloat16), ptb_ref[...],
                 preferred_element_type=jnp.float32)

    wv_parts = []
    wx_parts = []
    wy_parts = []
    wz_parts = []
    for u in range(8):
        m4 = u % 4
        hi = u // 4
        wv = None
        for t in range(16):
            c = m4 + 4 * t + 64 * hi
            s = -((cc + pp[c:c + 1, :]) - 2.0 * mm[:, c * 128:(c + 1) * 128])
            sx = x[c:c + 1, :]
            sy = y[c:c + 1, :]
            sz = z[c:c + 1, :]
            if wv is None:
                wv = s
                wx = jnp.broadcast_to(sx, s.shape)
                wy = jnp.broadcast_to(sy, s.shape)
                wz = jnp.broadcast_to(sz, s.shape)
            else:
                cond = s > wv
                wv = jnp.where(cond, s, wv)
                wx = jnp.where(cond, sx, wx)
                wy = jnp.where(cond, sy, wy)
                wz = jnp.where(cond, sz, wz)
        wv_parts.append(wv)
        wx_parts.append(wx)
        wy_parts.append(wy)
        wz_parts.append(wz)
    wv = jnp.concatenate(wv_parts, axis=1)
    wx = jnp.concatenate(wx_parts, axis=1)
    wy = jnp.concatenate(wy_parts, axis=1)
    wz = jnp.concatenate(wz_parts, axis=1)

    biota = jax.lax.broadcasted_iota(jnp.int32, (_ROWS, 1024), 1)
    kiota = jax.lax.broadcasted_iota(jnp.int32, (_ROWS, _K), 1)
    fx = jnp.zeros((_ROWS, _K), jnp.float32)
    fy = jnp.zeros((_ROWS, _K), jnp.float32)
    fz = jnp.zeros((_ROWS, _K), jnp.float32)
    for k in range(_K):
        m = jnp.max(wv, axis=1, keepdims=True)
        amin = jnp.min(jnp.where(wv == m, biota, 4096), axis=1, keepdims=True)
        sel = biota == amin
        gx = jnp.sum(jnp.where(sel, wx, 0.0), axis=1, keepdims=True)
        gy = jnp.sum(jnp.where(sel, wy, 0.0), axis=1, keepdims=True)
        gz = jnp.sum(jnp.where(sel, wz, 0.0), axis=1, keepdims=True)
        fx = jnp.where(kiota == k, gx, fx)
        fy = jnp.where(kiota == k, gy, fy)
        fz = jnp.where(kiota == k, gz, fz)
        wv = jnp.where(sel, _NEG_INF, wv)

    out_ref[0, :, :] = fx - cid
    out_ref[1, :, :] = fy - cid
    out_ref[2, :, :] = fz - cid
    out_ref[3, :, :] = fx
    out_ref[4, :, :] = fy
    out_ref[5, :, :] = fz


def _mlp_kernel(f_ref, w1_ref, b1_ref, s1_ref, t1_ref, m1_ref, v1_ref,
                w2_ref, b2_ref, s2_ref, t2_ref, m2_ref, v2_ref, out_ref):
    yy = jnp.dot(f_ref[...], w1_ref[...],
                 preferred_element_type=jnp.float32) + b1_ref[...]
    yy = (yy - m1_ref[...]) / jnp.sqrt(v1_ref[...] + _EPS) * s1_ref[...] + t1_ref[...]
    yy = jnp.maximum(yy, 0.0)
    yy = jnp.dot(yy, w2_ref[...],
                 preferred_element_type=jnp.float32) + b2_ref[...]
    yy = (yy - m2_ref[...]) / jnp.sqrt(v2_ref[...] + _EPS) * s2_ref[...] + t2_ref[...]
    out_ref[...] = jnp.maximum(yy, 0.0)


def kernel(points, random_key, W1, b1, scale1, bias1, mean1, var1,
           W2, b2, scale2, bias2, mean2, var2):
    pts = points[:, :3]
    px = pts[:, 0].reshape(128, 128)
    py = pts[:, 1].reshape(128, 128)
    pz = pts[:, 2].reshape(128, 128)

    id0 = jax.random.choice(random_key, _N, replace=False)
    id0 = jnp.asarray(id0, jnp.int32).reshape(1)

    ids = pl.pallas_call(
        _fps_kernel,
        in_specs=[pl.BlockSpec((128, 128), lambda: (0, 0))] * 3 +
                 [pl.BlockSpec(memory_space=pltpu.SMEM)],
        out_specs=pl.BlockSpec(memory_space=pltpu.SMEM),
        out_shape=jax.ShapeDtypeStruct((_M,), jnp.int32),
    )(px, py, pz, id0)

    ptb = jnp.zeros((128, _N), jnp.float32).at[:3, :].set(pts.T)
    ptb = ptb.astype(jnp.bfloat16)

    feats6 = pl.pallas_call(
        _knn_kernel,
        grid=(_CHUNKS,),
        in_specs=[pl.BlockSpec(memory_space=pltpu.SMEM),
                  pl.BlockSpec((128, 128), lambda i: (0, 0)),
                  pl.BlockSpec((128, 128), lambda i: (0, 0)),
                  pl.BlockSpec((128, 128), lambda i: (0, 0)),
                  pl.BlockSpec((128, _N), lambda i: (0, 0))],
        out_specs=pl.BlockSpec((6, _ROWS, _K), lambda i: (0, i, 0)),
        out_shape=jax.ShapeDtypeStruct((6, _M, _K), jnp.float32),
    )(ids, px, py, pz, ptb)

    feats = jnp.transpose(feats6, (1, 2, 0)).reshape(_M * _K, 6)
    feats = jnp.concatenate(
        [feats, jnp.zeros((_M * _K, 2), jnp.float32)], axis=1)

    w1p = jnp.zeros((8, 128), jnp.float32).at[:6, :].set(W1)
    row = lambda v: v.reshape(1, 128)

    BR = 1024
    out = pl.pallas_call(
        _mlp_kernel,
        grid=(_M * _K // BR,),
        in_specs=[pl.BlockSpec((BR, 8), lambda i: (i, 0))] +
                 [pl.BlockSpec((8, 128), lambda i: (0, 0))] +
                 [pl.BlockSpec((1, 128), lambda i: (0, 0))] * 5 +
                 [pl.BlockSpec((128, 128), lambda i: (0, 0))] +
                 [pl.BlockSpec((1, 128), lambda i: (0, 0))] * 5,
        out_specs=pl.BlockSpec((BR, 128), lambda i: (i, 0)),
        out_shape=jax.ShapeDtypeStruct((_M * _K, 128), jnp.float32),
    )(feats, w1p, row(b1), row(scale1), row(bias1), row(mean1), row(var1),
      W2, row(b2), row(scale2), row(bias2), row(mean2), row(var2))

    return out.reshape(_M, _K, 128)


# speedup vs baseline: 86.8441x; 1.0114x over previous
import jax
import jax.numpy as jnp
from jax.experimental import pallas as pl
from jax.experimental.pallas import tpu as pltpu

_N = 16384
_M = 256
_K = 32
_EPS = 1e-5
_ROWS = 32
_CHUNKS = _M // _ROWS

_NEG_INF = float('-inf')


def _fps_kernel(x_ref, y_ref, z_ref, id0_ref, ids_ref, dist_ref):
    liota = jax.lax.broadcasted_iota(jnp.int32, (1, 128), 1)
    sub8 = jax.lax.broadcasted_iota(jnp.int32, (8, 128), 0)
    lane8 = jax.lax.broadcasted_iota(jnp.int32, (8, 128), 1)

    def coords_of(idx):
        row = idx // 128
        sel = liota == (idx % 128)
        cx = jnp.sum(jnp.where(sel, x_ref[pl.ds(row, 1), :], 0.0))
        cy = jnp.sum(jnp.where(sel, y_ref[pl.ds(row, 1), :], 0.0))
        cz = jnp.sum(jnp.where(sel, z_ref[pl.ds(row, 1), :], 0.0))
        return cx, cy, cz

    id0 = id0_ref[0]
    ids_ref[0] = id0
    qx, qy, qz = coords_of(id0)
    dist_ref[...] = jnp.full((128, 128), jnp.inf, jnp.float32)

    def body(k, carry):
        qx, qy, qz, cur = carry
        row = cur // 128
        drow = dist_ref[pl.ds(row, 1), :]
        dist_ref[pl.ds(row, 1), :] = jnp.where(
            liota == (cur % 128), _NEG_INF, drow)

        macc = jnp.full((8, 128), _NEG_INF, jnp.float32)
        cidx = jnp.zeros((8, 128), jnp.int32)
        for c in range(16):
            sl = slice(c * 8, (c + 1) * 8)
            dx = x_ref[sl, :] - qx
            dy = y_ref[sl, :] - qy
            dz = z_ref[sl, :] - qz
            d = (dx * dx + dz * dz) + dy * dy
            nd = jnp.minimum(d, dist_ref[sl, :])
            dist_ref[sl, :] = nd
            upd = nd > macc
            macc = jnp.where(upd, nd, macc)
            cidx = jnp.where(upd, c, cidx)
        m = jnp.max(macc)
        linpos = (cidx * 8 + sub8) * 128 + lane8
        idx = jnp.min(jnp.where(macc == m, linpos, _N))
        ids_ref[k + 1] = idx
        nx, ny, nz = coords_of(idx)
        return nx, ny, nz, idx

    jax.lax.fori_loop(0, _M - 1, body, (qx, qy, qz, id0))


def _knn_kernel(ids_ref, x_ref, y_ref, z_ref, ptb_ref, out_ref):
    i = pl.program_id(0)
    x = x_ref[...]
    y = y_ref[...]
    z = z_ref[...]
    pp = (x * x + z * z) + y * y

    riota = jax.lax.broadcasted_iota(jnp.int32, (_ROWS, 1), 0)
    liota = jax.lax.broadcasted_iota(jnp.int32, (1, 128), 1)

    cx = jnp.zeros((_ROWS, 1), jnp.float32)
    cy = jnp.zeros((_ROWS, 1), jnp.float32)
    cz = jnp.zeros((_ROWS, 1), jnp.float32)
    cid = jnp.zeros((_ROWS, 1), jnp.float32)
    for r in range(_ROWS):
        idr = ids_ref[i * _ROWS + r]
        row = idr // 128
        lane = idr % 128
        sel = liota == lane
        gx = jnp.sum(jnp.where(sel, x_ref[pl.ds(row, 1), :], 0.0))
        gy = jnp.sum(jnp.where(sel, y_ref[pl.ds(row, 1), :], 0.0))
        gz = jnp.sum(jnp.where(sel, z_ref[pl.ds(row, 1), :], 0.0))
        cx = jnp.where(riota == r, gx, cx)
        cy = jnp.where(riota == r, gy, cy)
        cz = jnp.where(riota == r, gz, cz)
        cid = jnp.where(riota == r, idr.astype(jnp.float32), cid)

    cc = (cx * cx + cz * cz) + cy * cy

    cmat = jnp.where(liota == 0, cx, jnp.where(liota == 1, cy,
                     jnp.where(liota == 2, cz, 0.0)))
    mm = jnp.dot(cmat.astype(jnp.bfloat16), ptb_ref[...],
                 preferred_element_type=jnp.float32)

    wv_parts = []
    wx_parts = []
    wy_parts = []
    wz_parts = []
    for u in range(8):
        m4 = u % 4
        hi = u // 4
        wv = None
        for t in range(16):
            c = m4 + 4 * t + 64 * hi
            s = -((cc + pp[c:c + 1, :]) - 2.0 * mm[:, c * 128:(c + 1) * 128])
            sx = x[c:c + 1, :]
            sy = y[c:c + 1, :]
            sz = z[c:c + 1, :]
            if wv is None:
                wv = s
                wx = jnp.broadcast_to(sx, s.shape)
                wy = jnp.broadcast_to(sy, s.shape)
                wz = jnp.broadcast_to(sz, s.shape)
            else:
                cond = s > wv
                wv = jnp.where(cond, s, wv)
                wx = jnp.where(cond, sx, wx)
                wy = jnp.where(cond, sy, wy)
                wz = jnp.where(cond, sz, wz)
        wv_parts.append(wv)
        wx_parts.append(wx)
        wy_parts.append(wy)
        wz_parts.append(wz)
    wv = jnp.concatenate(wv_parts, axis=1)
    wx = jnp.concatenate(wx_parts, axis=1)
    wy = jnp.concatenate(wy_parts, axis=1)
    wz = jnp.concatenate(wz_parts, axis=1)

    biota = jax.lax.broadcasted_iota(jnp.int32, (_ROWS, 1024), 1)
    kiota = jax.lax.broadcasted_iota(jnp.int32, (_ROWS, _K), 1)
    fx = jnp.zeros((_ROWS, _K), jnp.float32)
    fy = jnp.zeros((_ROWS, _K), jnp.float32)
    fz = jnp.zeros((_ROWS, _K), jnp.float32)
    for k in range(_K):
        m = jnp.max(wv, axis=1, keepdims=True)
        amin = jnp.min(jnp.where(wv == m, biota, 4096), axis=1, keepdims=True)
        sel = biota == amin
        gx = jnp.sum(jnp.where(sel, wx, 0.0), axis=1, keepdims=True)
        gy = jnp.sum(jnp.where(sel, wy, 0.0), axis=1, keepdims=True)
        gz = jnp.sum(jnp.where(sel, wz, 0.0), axis=1, keepdims=True)
        fx = jnp.where(kiota == k, gx, fx)
        fy = jnp.where(kiota == k, gy, fy)
        fz = jnp.where(kiota == k, gz, fz)
        wv = jnp.where(sel, _NEG_INF, wv)

    out_ref[0, :, :] = fx - cid
    out_ref[1, :, :] = fy - cid
    out_ref[2, :, :] = fz - cid
    out_ref[3, :, :] = fx
    out_ref[4, :, :] = fy
    out_ref[5, :, :] = fz


def _mlp_kernel(f_ref, w1_ref, b1_ref, s1_ref, t1_ref, m1_ref, v1_ref,
                w2_ref, b2_ref, s2_ref, t2_ref, m2_ref, v2_ref, out_ref):
    yy = jnp.dot(f_ref[...], w1_ref[...],
                 preferred_element_type=jnp.float32) + b1_ref[...]
    yy = (yy - m1_ref[...]) / jnp.sqrt(v1_ref[...] + _EPS) * s1_ref[...] + t1_ref[...]
    yy = jnp.maximum(yy, 0.0)
    yy = jnp.dot(yy, w2_ref[...],
                 preferred_element_type=jnp.float32) + b2_ref[...]
    yy = (yy - m2_ref[...]) / jnp.sqrt(v2_ref[...] + _EPS) * s2_ref[...] + t2_ref[...]
    out_ref[...] = jnp.maximum(yy, 0.0)


def kernel(points, random_key, W1, b1, scale1, bias1, mean1, var1,
           W2, b2, scale2, bias2, mean2, var2):
    pts = points[:, :3]
    px = pts[:, 0].reshape(128, 128)
    py = pts[:, 1].reshape(128, 128)
    pz = pts[:, 2].reshape(128, 128)

    id0 = jax.random.choice(random_key, _N, replace=False)
    id0 = jnp.asarray(id0, jnp.int32).reshape(1)

    ids = pl.pallas_call(
        _fps_kernel,
        in_specs=[pl.BlockSpec((128, 128), lambda: (0, 0))] * 3 +
                 [pl.BlockSpec(memory_space=pltpu.SMEM)],
        out_specs=pl.BlockSpec(memory_space=pltpu.SMEM),
        out_shape=jax.ShapeDtypeStruct((_M,), jnp.int32),
        scratch_shapes=[pltpu.VMEM((128, 128), jnp.float32)],
    )(px, py, pz, id0)

    ptb = jnp.zeros((128, _N), jnp.float32).at[:3, :].set(pts.T)
    ptb = ptb.astype(jnp.bfloat16)

    feats6 = pl.pallas_call(
        _knn_kernel,
        grid=(_CHUNKS,),
        in_specs=[pl.BlockSpec(memory_space=pltpu.SMEM),
                  pl.BlockSpec((128, 128), lambda i: (0, 0)),
                  pl.BlockSpec((128, 128), lambda i: (0, 0)),
                  pl.BlockSpec((128, 128), lambda i: (0, 0)),
                  pl.BlockSpec((128, _N), lambda i: (0, 0))],
        out_specs=pl.BlockSpec((6, _ROWS, _K), lambda i: (0, i, 0)),
        out_shape=jax.ShapeDtypeStruct((6, _M, _K), jnp.float32),
    )(ids, px, py, pz, ptb)

    feats = jnp.transpose(feats6, (1, 2, 0)).reshape(_M * _K, 6)
    feats = jnp.concatenate(
        [feats, jnp.zeros((_M * _K, 2), jnp.float32)], axis=1)

    w1p = jnp.zeros((8, 128), jnp.float32).at[:6, :].set(W1)
    row = lambda v: v.reshape(1, 128)

    BR = 1024
    out = pl.pallas_call(
        _mlp_kernel,
        grid=(_M * _K // BR,),
        in_specs=[pl.BlockSpec((BR, 8), lambda i: (i, 0))] +
                 [pl.BlockSpec((8, 128), lambda i: (0, 0))] +
                 [pl.BlockSpec((1, 128), lambda i: (0, 0))] * 5 +
                 [pl.BlockSpec((128, 128), lambda i: (0, 0))] +
                 [pl.BlockSpec((1, 128), lambda i: (0, 0))] * 5,
        out_specs=pl.BlockSpec((BR, 128), lambda i: (i, 0)),
        out_shape=jax.ShapeDtypeStruct((_M * _K, 128), jnp.float32),
    )(feats, w1p, row(b1), row(scale1), row(bias1), row(mean1), row(var1),
      W2, row(b2), row(scale2), row(bias2), row(mean2), row(var2))

    return out.reshape(_M, _K, 128)
